# baseline (speedup 1.0000x reference)
import jax
import jax.numpy as jnp
from jax import lax
from jax.experimental import pallas as pl
from jax.experimental.pallas import tpu as pltpu
from jax.experimental.pallas import tpu_sc as plsc

_NC = 2
_NS = 16
_NW = _NC * _NS

_B = 4096 * 200
_D = 128
_L = 128
_K = 2
_G = _K * _L
_IDX_ROWS = _B // _L
_ROWS_PER_W = _IDX_ROWS // _NW
_GROUPS = _ROWS_PER_W // _K
_PAIRS = _GROUPS // 2


def _emb_body(idx_hbm, table_hbm, out_hbm,
              idx0, idx1, rows0, rows1, table_sp, store_sp, gsem0, gsem1):
    idxs = (idx0, idx1)
    rows = (rows0, rows1)
    gsems = (gsem0, gsem1)

    sid = lax.axis_index("s")
    wid = sid * _NC + lax.axis_index("c")
    idx_row0 = wid * _ROWS_PER_W
    out_row0 = idx_row0 * _L

    @pl.when(sid == 0)
    def _stage():
        pltpu.sync_copy(table_hbm, table_sp)

    plsc.subcore_barrier()

    def fetch(b, g):
        pltpu.sync_copy(idx_hbm.at[pl.ds(idx_row0 + g * _K, _K)], idxs[b])
        for j in range(_K):
            pltpu.async_copy(
                table_sp.at[idxs[b].at[j]],
                rows[b].at[pl.ds(j * _L, _L)],
                gsems[b],
            )

    def drain_store(b, g):
        for j in range(_K):
            pltpu.make_async_copy(
                table_sp.at[idxs[b].at[j]],
                rows[b].at[pl.ds(j * _L, _L)],
                gsems[b],
            ).wait()
        pltpu.sync_copy(rows[b], store_sp.at[sid])
        pltpu.sync_copy(store_sp.at[sid],
                        out_hbm.at[pl.ds(out_row0 + g * _G, _G)])

    fetch(0, 0)
    fetch(1, 1)

    def step(t, carry):
        drain_store(0, 2 * t)
        fetch(0, 2 * t + 2)
        drain_store(1, 2 * t + 1)
        fetch(1, 2 * t + 3)
        return carry

    lax.fori_loop(0, _PAIRS - 1, step, 0)
    drain_store(0, 2 * (_PAIRS - 1))
    drain_store(1, 2 * (_PAIRS - 1) + 1)


_emb_kernel = pl.kernel(
    _emb_body,
    out_type=jax.ShapeDtypeStruct((_B, _D), jnp.float32),
    mesh=plsc.VectorSubcoreMesh(
        core_axis_name="c", subcore_axis_name="s",
        num_cores=_NC, num_subcores=_NS,
    ),
    scratch_types=[
        pltpu.VMEM((_K, _L), jnp.int32),
        pltpu.VMEM((_K, _L), jnp.int32),
        pltpu.VMEM((_G, _D), jnp.float32),
        pltpu.VMEM((_G, _D), jnp.float32),
        pltpu.VMEM_SHARED((1001, _D), jnp.float32),
        pltpu.VMEM_SHARED((_NS, _G, _D), jnp.float32),
        pltpu.SemaphoreType.DMA,
        pltpu.SemaphoreType.DMA,
    ],
)


@jax.jit
def kernel(time_features, table):
    bsz, seq = time_features.shape
    idx = time_features.reshape(_IDX_ROWS, _L).astype(jnp.int32)
    table = table.at[0].set(0.0)
    out = _emb_kernel(idx, table)
    return out.reshape(bsz, seq, _D)


# speedup vs baseline: 2.6575x; 2.6575x over previous
import jax
import jax.numpy as jnp
from jax import lax
from jax.experimental import pallas as pl
from jax.experimental.pallas import tpu as pltpu
from jax.experimental.pallas import tpu_sc as plsc

_NC = 2
_NS = 16
_NW = _NC * _NS

_B = 4096 * 200
_D = 128
_L = 128
_K = 2
_G = _K * _L
_IDX_ROWS = _B // _L
_ROWS_PER_W = _IDX_ROWS // _NW
_GROUPS = _ROWS_PER_W // _K
_PAIRS = _GROUPS // 2


def _emb_body(idx_hbm, table_hbm, out_hbm,
              idx_all, rows0, rows1, table_sp, gsem0, gsem1):
    rows = (rows0, rows1)
    gsems = (gsem0, gsem1)

    sid = lax.axis_index("s")
    wid = sid * _NC + lax.axis_index("c")
    idx_row0 = wid * _ROWS_PER_W
    out_row0 = idx_row0 * _L

    @pl.when(sid == 0)
    def _stage():
        pltpu.sync_copy(table_hbm, table_sp)

    pltpu.sync_copy(idx_hbm.at[pl.ds(idx_row0, _ROWS_PER_W)], idx_all)

    plsc.subcore_barrier()

    def fetch(b, g):
        for j in range(_K):
            pltpu.async_copy(
                table_sp.at[idx_all.at[g * _K + j]],
                rows[b].at[pl.ds(j * _L, _L)],
                gsems[b],
            )

    def drain_store(b, g):
        for j in range(_K):
            pltpu.make_async_copy(
                table_sp.at[idx_all.at[g * _K + j]],
                rows[b].at[pl.ds(j * _L, _L)],
                gsems[b],
            ).wait()
        pltpu.sync_copy(rows[b], out_hbm.at[pl.ds(out_row0 + g * _G, _G)])

    fetch(0, 0)
    fetch(1, 1)

    def step(t, carry):
        drain_store(0, 2 * t)
        fetch(0, 2 * t + 2)
        drain_store(1, 2 * t + 1)
        fetch(1, 2 * t + 3)
        return carry

    lax.fori_loop(0, _PAIRS - 1, step, 0)
    drain_store(0, 2 * (_PAIRS - 1))
    drain_store(1, 2 * (_PAIRS - 1) + 1)


_emb_kernel = pl.kernel(
    _emb_body,
    out_type=jax.ShapeDtypeStruct((_B, _D), jnp.float32),
    mesh=plsc.VectorSubcoreMesh(
        core_axis_name="c", subcore_axis_name="s",
        num_cores=_NC, num_subcores=_NS,
    ),
    scratch_types=[
        pltpu.VMEM((_ROWS_PER_W, _L), jnp.int32),
        pltpu.VMEM((_G, _D), jnp.float32),
        pltpu.VMEM((_G, _D), jnp.float32),
        pltpu.VMEM_SHARED((1001, _D), jnp.float32),
        pltpu.SemaphoreType.DMA,
        pltpu.SemaphoreType.DMA,
    ],
)


@jax.jit
def kernel(time_features, table):
    bsz, seq = time_features.shape
    idx = time_features.reshape(_IDX_ROWS, _L).astype(jnp.int32)
    table = table.at[0].set(0.0)
    out = _emb_kernel(idx, table)
    return out.reshape(bsz, seq, _D)
